# Initial kernel scaffold; baseline (speedup 1.0000x reference)
import jax
import jax.numpy as jnp
from jax.experimental import pallas as pl


def kernel(x, edge_index, W1, a_src1, a_dst1, b1, W2, a_src2, a_dst2, b2, Wm1, bm1, Wm2, bm2):
    raise NotImplementedError("write your pallas kernel here")


# speedup vs baseline: 30.9783x; 30.9783x over previous
import functools

import jax
import jax.numpy as jnp
from jax import lax
from jax.experimental import pallas as pl
from jax.experimental.pallas import tpu as pltpu
from jax.experimental.pallas import tpu_sc as plsc

N = 10000
E = 320000
F_IN = 128
HID = 128
HEADS = 8
OUT = 16
MLP_H = 256

NC = 2
NS = 16
NW = NC * NS
EPW = E // NW
CH = 80
NCHUNK = EPW // CH
RB = 400
NRB = N // RB

BM = 1000
GRID = N // BM

_mesh = plsc.VectorSubcoreMesh(core_axis_name="c", subcore_axis_name="s")


_DNUMS = lax.GatherDimensionNumbers(
    offset_dims=(), collapsed_slice_dims=(0,), start_index_map=(0,))


def _lane_gather(x, idx):
    return lax.gather(x, idx[:, None], _DNUMS, (1,),
                      mode=lax.GatherScatterMode.PROMISE_IN_BOUNDS)


def _splat(vec, j):
    return _lane_gather(vec, jnp.full((16,), j, jnp.int32))


def _lane_allsum(x):
    lanes = lax.iota(jnp.int32, 16)
    for sh in (8, 4, 2, 1):
        x = x + _lane_gather(x, lanes ^ sh)
    return x


def _k1_body(x_ref, w_ref, a_ref, ar_ref, h_ref, e_ref, er_ref):
    h = jnp.dot(x_ref[...], w_ref[...], preferred_element_type=jnp.float32)
    h_ref[...] = h
    e_ref[...] = jnp.dot(h, a_ref[...], preferred_element_type=jnp.float32)
    er_ref[...] = jnp.dot(h, ar_ref[...], preferred_element_type=jnp.float32)


def _merge(acc_ref, den_ref, h_ref, e_ref, b_ref, erep_ref):
    e = e_ref[...]
    v = e[:, :HEADS] + e[:, HEADS:]
    v = jnp.where(v >= 0.0, v, 0.2 * v)
    exs = jnp.exp(v)
    erep = erep_ref[...]
    exs_w = jnp.dot(exs, erep, preferred_element_type=jnp.float32)
    num = acc_ref[0] + acc_ref[1] + h_ref[...] * exs_w
    den8 = den_ref[0][:, :HEADS] + den_ref[1][:, :HEADS] + exs
    den = jnp.dot(den8, erep, preferred_element_type=jnp.float32) + 1e-16
    return jnp.maximum(num / den + b_ref[...], 0.0)


def _k2_body(acc_ref, den_ref, h_ref, e_ref, b_ref, erep_ref,
             w_ref, a_ref, ar_ref, h2_ref, e2_ref, e2r_ref):
    x1 = _merge(acc_ref, den_ref, h_ref, e_ref, b_ref, erep_ref)
    h2 = jnp.dot(x1, w_ref[...], preferred_element_type=jnp.float32)
    h2_ref[...] = h2
    e2_ref[...] = jnp.dot(h2, a_ref[...], preferred_element_type=jnp.float32)
    e2r_ref[...] = jnp.dot(h2, ar_ref[...], preferred_element_type=jnp.float32)


def _k3_body(acc_ref, den_ref, h_ref, e_ref, b_ref, erep_ref,
             wma_ref, wmb_ref, bm1_ref, p_ref, q_ref):
    x2 = _merge(acc_ref, den_ref, h_ref, e_ref, b_ref, erep_ref)
    p_ref[...] = jnp.dot(x2, wma_ref[...], preferred_element_type=jnp.float32)
    q_ref[...] = (jnp.dot(x2, wmb_ref[...], preferred_element_type=jnp.float32)
                  + bm1_ref[...])


def _row_spec(w):
    return pl.BlockSpec((BM, w), lambda i: (i, 0))


def _full_spec(shape):
    return pl.BlockSpec(shape, lambda i: tuple(0 for _ in shape))


def _tc_k1(x, w1, a1, a1r):
    return pl.pallas_call(
        _k1_body,
        grid=(GRID,),
        in_specs=[_row_spec(F_IN), _full_spec((F_IN, HID)),
                  _full_spec((HID, 16)), _full_spec((HID, 16))],
        out_specs=[_row_spec(HID), _row_spec(16), _row_spec(16)],
        out_shape=[jax.ShapeDtypeStruct((N, HID), jnp.float32),
                   jax.ShapeDtypeStruct((N, 16), jnp.float32),
                   jax.ShapeDtypeStruct((N, 16), jnp.float32)],
    )(x, w1, a1, a1r)


def _tc_k2(acc, den, h, esd, b, erep, w2, a2, a2r):
    return pl.pallas_call(
        _k2_body,
        grid=(GRID,),
        in_specs=[pl.BlockSpec((2, BM, HID), lambda i: (0, i, 0)),
                  pl.BlockSpec((2, BM, 16), lambda i: (0, i, 0)),
                  _row_spec(HID), _row_spec(16), _full_spec((1, HID)),
                  _full_spec((HEADS, HID)), _full_spec((HID, HID)),
                  _full_spec((HID, 16)), _full_spec((HID, 16))],
        out_specs=[_row_spec(HID), _row_spec(16), _row_spec(16)],
        out_shape=[jax.ShapeDtypeStruct((N, HID), jnp.float32),
                   jax.ShapeDtypeStruct((N, 16), jnp.float32),
                   jax.ShapeDtypeStruct((N, 16), jnp.float32)],
    )(acc, den, h, esd, b, erep, w2, a2, a2r)


def _tc_k3(acc, den, h, esd, b, erep, wma, wmb, bm1):
    return pl.pallas_call(
        _k3_body,
        grid=(GRID,),
        in_specs=[pl.BlockSpec((2, BM, HID), lambda i: (0, i, 0)),
                  pl.BlockSpec((2, BM, 16), lambda i: (0, i, 0)),
                  _row_spec(HID), _row_spec(16), _full_spec((1, HID)),
                  _full_spec((HEADS, HID)), _full_spec((HID, MLP_H)),
                  _full_spec((HID, MLP_H)), _full_spec((1, MLP_H))],
        out_specs=[_row_spec(MLP_H), _row_spec(MLP_H)],
        out_shape=[jax.ShapeDtypeStruct((N, MLP_H), jnp.float32),
                   jax.ShapeDtypeStruct((N, MLP_H), jnp.float32)],
    )(acc, den, h, esd, b, erep, wma, wmb, bm1)


def _sc_layer_body(esd_hbm, esdr_hbm, h_hbm, src_hbm, dst_hbm, z128_hbm,
                   z16_hbm, acc_hbm, den_hbm,
                   idx_s, idx_d, gs, gd, hrow, msg, exb, acc_sh, den_sh, sem):
    c = lax.axis_index("c")
    s = lax.axis_index("s")
    wid = s * NC + c

    for k in range((NRB + NS - 1) // NS):
        blk = s + k * NS

        @pl.when(blk < NRB)
        def _init():
            r0 = blk * RB
            pltpu.sync_copy(z128_hbm.at[pl.ds(r0, RB)],
                            acc_sh.at[pl.ds(r0, RB)])
            pltpu.sync_copy(z16_hbm.at[pl.ds(r0, RB)],
                            den_sh.at[pl.ds(r0, RB)])

    plsc.subcore_barrier()

    base = wid * EPW

    def chunk_body(ci, _):
        off = base + ci * CH
        pltpu.sync_copy(src_hbm.at[pl.ds(off, CH)], idx_s)
        pltpu.sync_copy(dst_hbm.at[pl.ds(off, CH)], idx_d)
        pltpu.async_copy(esd_hbm.at[idx_s], gs, sem).wait()
        pltpu.async_copy(esdr_hbm.at[idx_d], gd, sem).wait()
        pltpu.async_copy(h_hbm.at[idx_s], hrow, sem).wait()

        def edge_body(e, carry):
            v = gs[e] + gd[e]
            v = jnp.where(v >= 0.0, v, 0.2 * v)
            ex = jnp.exp(v)
            exb[e] = ex
            for j in range(HEADS):
                sp = _splat(ex, j)
                msg[e, pl.ds(j * OUT, OUT)] = hrow[e, pl.ds(j * OUT, OUT)] * sp
            return carry

        lax.fori_loop(0, CH, edge_body, 0)
        pltpu.sync_copy(msg, acc_sh.at[idx_d], add=True)
        pltpu.sync_copy(exb, den_sh.at[idx_d], add=True)
        return _

    lax.fori_loop(0, NCHUNK, chunk_body, 0)
    plsc.subcore_barrier()

    for k in range((NRB + NS - 1) // NS):
        blk = s + k * NS

        @pl.when(blk < NRB)
        def _out():
            r0 = blk * RB
            pltpu.sync_copy(acc_sh.at[pl.ds(r0, RB)],
                            acc_hbm.at[c, pl.ds(r0, RB)])
            pltpu.sync_copy(den_sh.at[pl.ds(r0, RB)],
                            den_hbm.at[c, pl.ds(r0, RB)])


@functools.partial(
    pl.kernel,
    out_type=[jax.ShapeDtypeStruct((NC, N, HID), jnp.float32),
              jax.ShapeDtypeStruct((NC, N, 16), jnp.float32)],
    mesh=_mesh,
    compiler_params=pltpu.CompilerParams(use_tc_tiling_on_sc=False),
    scratch_types=[
        pltpu.VMEM((CH,), jnp.int32),
        pltpu.VMEM((CH,), jnp.int32),
        pltpu.VMEM((CH, 16), jnp.float32),
        pltpu.VMEM((CH, 16), jnp.float32),
        pltpu.VMEM((CH, HID), jnp.float32),
        pltpu.VMEM((CH, HID), jnp.float32),
        pltpu.VMEM((CH, 16), jnp.float32),
        pltpu.VMEM_SHARED((N, HID), jnp.float32),
        pltpu.VMEM_SHARED((N, 16), jnp.float32),
        pltpu.SemaphoreType.DMA,
    ],
)
def _sc_layer(esd, esdr, h, src, dst, z128, z16, acc, den, *scratch):
    _sc_layer_body(esd, esdr, h, src, dst, z128, z16, acc, den, *scratch)


def _sc_mlp_body(p_hbm, q_hbm, src_hbm, dst_hbm, wm2_hbm, bm2_hbm, pred_hbm,
                 idx_s, idx_d, pbuf, qbuf, wv, bv, predb, sem):
    c = lax.axis_index("c")
    s = lax.axis_index("s")
    wid = s * NC + c
    base = wid * EPW

    pltpu.sync_copy(wm2_hbm, wv)
    pltpu.sync_copy(bm2_hbm, bv)

    def chunk_body(ci, _):
        off = base + ci * CH
        pltpu.sync_copy(src_hbm.at[pl.ds(off, CH)], idx_s)
        pltpu.sync_copy(dst_hbm.at[pl.ds(off, CH)], idx_d)
        pltpu.async_copy(p_hbm.at[idx_s], pbuf, sem).wait()
        pltpu.async_copy(q_hbm.at[idx_d], qbuf, sem).wait()

        lanes = lax.iota(jnp.int32, 16)

        def grp_body(g, _g):
            def edge_body(k, res):
                e = g * 16 + k
                acc = jnp.zeros((16,), jnp.float32)
                for j in range(MLP_H // 16):
                    hj = jnp.maximum(
                        pbuf[e, pl.ds(j * 16, 16)] + qbuf[e, pl.ds(j * 16, 16)],
                        0.0)
                    acc = acc + hj * wv[pl.ds(j * 16, 16)]
                sfull = _lane_allsum(acc)
                return jnp.where(lanes == k, sfull, res)

            res = lax.fori_loop(0, 16, edge_body, jnp.zeros((16,), jnp.float32))
            predb[pl.ds(g * 16, 16)] = res + bv[...]
            return _g

        lax.fori_loop(0, CH // 16, grp_body, 0)
        pltpu.sync_copy(predb, pred_hbm.at[pl.ds(off, CH)])
        return _

    lax.fori_loop(0, NCHUNK, chunk_body, 0)


@functools.partial(
    pl.kernel,
    out_type=jax.ShapeDtypeStruct((E,), jnp.float32),
    mesh=_mesh,
    scratch_types=[
        pltpu.VMEM((CH,), jnp.int32),
        pltpu.VMEM((CH,), jnp.int32),
        pltpu.VMEM((CH, MLP_H), jnp.float32),
        pltpu.VMEM((CH, MLP_H), jnp.float32),
        pltpu.VMEM((MLP_H,), jnp.float32),
        pltpu.VMEM((16,), jnp.float32),
        pltpu.VMEM((CH,), jnp.float32),
        pltpu.SemaphoreType.DMA,
    ],
)
def _sc_mlp(p, q, src, dst, wm2, bm2, pred, *scratch):
    _sc_mlp_body(p, q, src, dst, wm2, bm2, pred, *scratch)


def _pack_a(a_src, a_dst):
    eye = jnp.eye(HEADS, dtype=jnp.float32)
    a_es = (a_src[:, :, None] * eye[:, None, :]).reshape(HID, HEADS)
    a_ed = (a_dst[:, :, None] * eye[:, None, :]).reshape(HID, HEADS)
    return (jnp.concatenate([a_es, a_ed], axis=1),
            jnp.concatenate([a_ed, a_es], axis=1))


def kernel(x, edge_index, W1, a_src1, a_dst1, b1, W2, a_src2, a_dst2, b2,
           Wm1, bm1, Wm2, bm2):
    src = edge_index[0]
    dst = edge_index[1]
    a1, a1r = _pack_a(a_src1, a_dst1)
    a2, a2r = _pack_a(a_src2, a_dst2)
    erep = jnp.kron(jnp.eye(HEADS, dtype=jnp.float32),
                    jnp.ones((1, OUT), jnp.float32))
    z128 = jnp.zeros((N, HID), jnp.float32)
    z16 = jnp.zeros((N, 16), jnp.float32)

    h1, esd1, esd1r = _tc_k1(x, W1, a1, a1r)
    acc1, den1 = _sc_layer(esd1, esd1r, h1, src, dst, z128, z16)
    h2, esd2, esd2r = _tc_k2(acc1, den1, h1, esd1, b1.reshape(1, HID), erep,
                             W2, a2, a2r)
    acc2, den2 = _sc_layer(esd2, esd2r, h2, src, dst, z128, z16)
    p, q = _tc_k3(acc2, den2, h2, esd2, b2.reshape(1, HID), erep,
                  Wm1[:HID], Wm1[HID:], bm1.reshape(1, MLP_H))
    pred = _sc_mlp(p, q, src, dst, Wm2.reshape(-1),
                   jnp.broadcast_to(bm2, (16,)))
    return pred
